# baseline (speedup 1.0000x reference)
import functools

import jax
import jax.numpy as jnp
from jax import lax
from jax.experimental import pallas as pl
from jax.experimental.pallas import tpu as pltpu
from jax.experimental.pallas import tpu_sc as plsc

B = 16384
L = 26
D = 64
NL = 16
NC = 2
NS = 16
NW = NC * NS
BPW = B // NW
LP = 32
G = 8
GL = G * L
GH = GL // 2
NG = BPW // G
DSL = D // NL


NBUF = 2


def _body(idx_hbm, w_hbm, table_hbm, out_hbm,
          idx_v, w_v, out_v, bufs, sems):
    c = lax.axis_index("c")
    s = lax.axis_index("s")
    wid = s * NC + c
    base = wid * BPW

    pltpu.sync_copy(idx_hbm.at[pl.ds(base * L, BPW * L)], idx_v)
    pltpu.sync_copy(w_hbm.at[pl.ds(base * LP, BPW * LP)], w_v)

    def start(step, buf, sem):
        pltpu.make_async_copy(
            table_hbm.at[idx_v.at[pl.ds(step * GL, GH)]],
            buf.at[pl.ds(0, GH)], sem).start()
        pltpu.make_async_copy(
            table_hbm.at[idx_v.at[pl.ds(step * GL + GH, GH)]],
            buf.at[pl.ds(GH, GH)], sem).start()

    def wait(buf, sem):
        pltpu.make_async_copy(table_hbm.at[idx_v.at[pl.ds(0, GL)]],
                              buf, sem).wait()

    def compute(step, buf):
        for g in range(G):
            prow = step * G + g
            wv0 = w_v[pl.ds(prow * LP, NL)]
            wv1 = w_v[pl.ds(prow * LP + NL, NL)]
            accs = [jnp.zeros((NL,), jnp.float32) for _ in range(DSL)]
            for l in range(L):
                wl = wv0[l] if l < NL else wv1[l - NL]
                w = lax.broadcast(wl, (NL,))
                r = g * L + l
                for k in range(DSL):
                    accs[k] = accs[k] + w * buf[r, pl.ds(k * NL, NL)]
            for k in range(DSL):
                out_v[prow, pl.ds(k * NL, NL)] = accs[k]

    for k in range(NBUF - 1):
        start(k, bufs[k], sems[k])

    def loop_body(i, carry):
        for k in range(NBUF):
            step = NBUF * i + k
            nxt = step + NBUF - 1

            @pl.when(nxt < NG)
            def _():
                start(nxt, bufs[(k + NBUF - 1) % NBUF], sems[(k + NBUF - 1) % NBUF])

            wait(bufs[k], sems[k])
            compute(step, bufs[k])
        return carry

    lax.fori_loop(0, NG // NBUF, loop_body, 0)

    pltpu.sync_copy(out_v, out_hbm.at[pl.ds(base, BPW)])


V = 1000000
TBLK = 32768
NT = (V + TBLK - 1) // TBLK


VP = NT * TBLK


def _transpose_body(tT_ref, out_ref):
    m = tT_ref[...]
    half = TBLK // 2
    out_ref[:, :D] = jnp.transpose(m[:, :half], (1, 0))
    out_ref[:, D:] = jnp.transpose(m[:, half:], (1, 0))


def _linearize_table(table):
    t128 = pl.pallas_call(
        _transpose_body,
        grid=(NT,),
        in_specs=[pl.BlockSpec((D, TBLK), lambda i: (0, i))],
        out_specs=pl.BlockSpec((TBLK // 2, 2 * D), lambda i: (i, 0)),
        out_shape=jax.ShapeDtypeStruct((VP // 2, 2 * D), jnp.float32),
    )(table.T)
    return t128.reshape(VP, D)


def _remap_indices(idx):
    pos = idx % TBLK
    half = TBLK // 2
    return idx - pos + jnp.where(pos < half, 2 * pos, 2 * (pos - half) + 1)


@jax.jit
def kernel(feature_indices, feature_weights, table):
    idx = _remap_indices(feature_indices.reshape(-1).astype(jnp.int32))
    w = jnp.pad(feature_weights, ((0, 0), (0, LP - L))).reshape(-1)
    table = _linearize_table(table)

    mesh = plsc.VectorSubcoreMesh(core_axis_name="c", subcore_axis_name="s")
    f = pl.kernel(
        _body,
        out_type=jax.ShapeDtypeStruct((B, D), jnp.float32),
        mesh=mesh,
        compiler_params=pltpu.CompilerParams(use_tc_tiling_on_sc=False),
        scratch_types=[
            pltpu.VMEM((BPW * L,), jnp.int32),
            pltpu.VMEM((BPW * LP,), jnp.float32),
            pltpu.VMEM((BPW, D), jnp.float32),
            [pltpu.VMEM((GL, D), jnp.float32) for _ in range(NBUF)],
            [pltpu.SemaphoreType.DMA for _ in range(NBUF)],
        ],
    )
    return f(idx, w, table)


# speedup vs baseline: 1.0035x; 1.0035x over previous
import functools

import jax
import jax.numpy as jnp
from jax import lax
from jax.experimental import pallas as pl
from jax.experimental.pallas import tpu as pltpu
from jax.experimental.pallas import tpu_sc as plsc

B = 16384
L = 26
D = 64
NL = 16
NC = 2
NS = 16
NW = NC * NS
BPW = B // NW
LP = 32
G = 4
GL = G * L
NG = BPW // G
DSL = D // NL


NBUF = 2


def _body(idx_hbm, w_hbm, table_hbm, out_hbm,
          idx_v, w_v, out_v, bufs, sems):
    c = lax.axis_index("c")
    s = lax.axis_index("s")
    wid = s * NC + c
    base = wid * BPW

    pltpu.sync_copy(idx_hbm.at[pl.ds(base * L, BPW * L)], idx_v)
    pltpu.sync_copy(w_hbm.at[pl.ds(base * LP, BPW * LP)], w_v)

    def start(step, buf, sem):
        pltpu.make_async_copy(
            table_hbm.at[idx_v.at[pl.ds(step * GL, GL)]], buf, sem).start()

    def wait(buf, sem):
        pltpu.make_async_copy(table_hbm.at[idx_v.at[pl.ds(0, GL)]],
                              buf, sem).wait()

    def compute(step, buf):
        for g in range(G):
            prow = step * G + g
            wv0 = w_v[pl.ds(prow * LP, NL)]
            wv1 = w_v[pl.ds(prow * LP + NL, NL)]
            accs = [jnp.zeros((NL,), jnp.float32) for _ in range(DSL)]
            for l in range(L):
                wl = wv0[l] if l < NL else wv1[l - NL]
                w = lax.broadcast(wl, (NL,))
                r = g * L + l
                for k in range(DSL):
                    accs[k] = accs[k] + w * buf[r, pl.ds(k * NL, NL)]
            for k in range(DSL):
                out_v[prow, pl.ds(k * NL, NL)] = accs[k]

    for k in range(NBUF - 1):
        start(k, bufs[k], sems[k])

    def loop_body(i, carry):
        for k in range(NBUF):
            step = NBUF * i + k
            nxt = step + NBUF - 1

            @pl.when(nxt < NG)
            def _():
                start(nxt, bufs[(k + NBUF - 1) % NBUF], sems[(k + NBUF - 1) % NBUF])

            wait(bufs[k], sems[k])
            compute(step, bufs[k])
        return carry

    lax.fori_loop(0, NG // NBUF, loop_body, 0)

    pltpu.sync_copy(out_v, out_hbm.at[pl.ds(base, BPW)])


V = 1000000
TBLK = 16384
NT = (V + TBLK - 1) // TBLK


VP = NT * TBLK


def _transpose_body(tT_ref, out_ref):
    m = tT_ref[...]
    half = TBLK // 2
    out_ref[:, :D] = jnp.transpose(m[:, :half], (1, 0))
    out_ref[:, D:] = jnp.transpose(m[:, half:], (1, 0))


def _linearize_table(table):
    t128 = pl.pallas_call(
        _transpose_body,
        grid=(NT,),
        in_specs=[pl.BlockSpec((D, TBLK), lambda i: (0, i))],
        out_specs=pl.BlockSpec((TBLK // 2, 2 * D), lambda i: (i, 0)),
        out_shape=jax.ShapeDtypeStruct((VP // 2, 2 * D), jnp.float32),
    )(table.T)
    return t128.reshape(VP, D)


def _remap_indices(idx):
    pos = idx % TBLK
    half = TBLK // 2
    return idx - pos + jnp.where(pos < half, 2 * pos, 2 * (pos - half) + 1)


@jax.jit
def kernel(feature_indices, feature_weights, table):
    idx = _remap_indices(feature_indices.reshape(-1).astype(jnp.int32))
    w = jnp.pad(feature_weights, ((0, 0), (0, LP - L))).reshape(-1)
    table = _linearize_table(table)

    mesh = plsc.VectorSubcoreMesh(core_axis_name="c", subcore_axis_name="s")
    f = pl.kernel(
        _body,
        out_type=jax.ShapeDtypeStruct((B, D), jnp.float32),
        mesh=mesh,
        compiler_params=pltpu.CompilerParams(use_tc_tiling_on_sc=False),
        scratch_types=[
            pltpu.VMEM((BPW * L,), jnp.int32),
            pltpu.VMEM((BPW * LP,), jnp.float32),
            pltpu.VMEM((BPW, D), jnp.float32),
            [pltpu.VMEM((GL, D), jnp.float32) for _ in range(NBUF)],
            [pltpu.SemaphoreType.DMA for _ in range(NBUF)],
        ],
    )
    return f(idx, w, table)


# speedup vs baseline: 1.0345x; 1.0309x over previous
import functools

import jax
import jax.numpy as jnp
from jax import lax
from jax.experimental import pallas as pl
from jax.experimental.pallas import tpu as pltpu
from jax.experimental.pallas import tpu_sc as plsc

B = 16384
L = 26
D = 64
NL = 16
NC = 2
NS = 16
NW = NC * NS
BPW = B // NW
LP = 32
G = 4
GL = G * L
NG = BPW // G
DSL = D // NL


NBUF = 2


def _body(idx_hbm, w_hbm, table_hbm, out_hbm,
          idx_v, w_v, out_v, bufs, sems):
    c = lax.axis_index("c")
    s = lax.axis_index("s")
    wid = s * NC + c
    base = wid * BPW

    pltpu.sync_copy(idx_hbm.at[pl.ds(base * L, BPW * L)], idx_v)
    pltpu.sync_copy(w_hbm.at[pl.ds(base * LP, BPW * LP)], w_v)

    def start(step, buf, sem):
        pltpu.make_async_copy(
            table_hbm.at[idx_v.at[pl.ds(step * GL, GL)]], buf, sem).start()

    def wait(buf, sem):
        pltpu.make_async_copy(table_hbm.at[idx_v.at[pl.ds(0, GL)]],
                              buf, sem).wait()

    def compute(step, buf):
        for g in range(G):
            prow = step * G + g
            wv0 = w_v[pl.ds(prow * LP, NL)]
            wv1 = w_v[pl.ds(prow * LP + NL, NL)]
            accs = [jnp.zeros((NL,), jnp.float32) for _ in range(DSL)]
            for l in range(L):
                wl = wv0[l] if l < NL else wv1[l - NL]
                w = lax.broadcast(wl, (NL,))
                r = g * L + l
                for k in range(DSL):
                    accs[k] = accs[k] + w * buf[r, pl.ds(k * NL, NL)]
            for k in range(DSL):
                out_v[prow, pl.ds(k * NL, NL)] = accs[k]

    for k in range(NBUF - 1):
        start(k, bufs[k], sems[k])

    def loop_body(i, carry):
        for k in range(NBUF):
            step = NBUF * i + k
            nxt = step + NBUF - 1

            @pl.when(nxt < NG)
            def _():
                start(nxt, bufs[(k + NBUF - 1) % NBUF], sems[(k + NBUF - 1) % NBUF])

            wait(bufs[k], sems[k])
            compute(step, bufs[k])
        return carry

    lax.fori_loop(0, NG // NBUF, loop_body, 0)

    pltpu.sync_copy(out_v, out_hbm.at[pl.ds(base, BPW)])


V = 1000000
TBLK = 40960
NT = (V + TBLK - 1) // TBLK


VP = NT * TBLK


def _transpose_body(tT_ref, out_ref):
    m = tT_ref[...]
    half = TBLK // 2
    out_ref[:, :D] = jnp.transpose(m[:, :half], (1, 0))
    out_ref[:, D:] = jnp.transpose(m[:, half:], (1, 0))


def _linearize_table(table):
    t128 = pl.pallas_call(
        _transpose_body,
        grid=(NT,),
        in_specs=[pl.BlockSpec((D, TBLK), lambda i: (0, i))],
        out_specs=pl.BlockSpec((TBLK // 2, 2 * D), lambda i: (i, 0)),
        out_shape=jax.ShapeDtypeStruct((VP // 2, 2 * D), jnp.float32),
    )(table.T)
    return t128.reshape(VP, D)


def _remap_indices(idx):
    pos = idx % TBLK
    half = TBLK // 2
    return idx - pos + jnp.where(pos < half, 2 * pos, 2 * (pos - half) + 1)


@jax.jit
def kernel(feature_indices, feature_weights, table):
    idx = _remap_indices(feature_indices.reshape(-1).astype(jnp.int32))
    w = jnp.pad(feature_weights, ((0, 0), (0, LP - L))).reshape(-1)
    table = _linearize_table(table)

    mesh = plsc.VectorSubcoreMesh(core_axis_name="c", subcore_axis_name="s")
    f = pl.kernel(
        _body,
        out_type=jax.ShapeDtypeStruct((B, D), jnp.float32),
        mesh=mesh,
        compiler_params=pltpu.CompilerParams(use_tc_tiling_on_sc=False),
        scratch_types=[
            pltpu.VMEM((BPW * L,), jnp.int32),
            pltpu.VMEM((BPW * LP,), jnp.float32),
            pltpu.VMEM((BPW, D), jnp.float32),
            [pltpu.VMEM((GL, D), jnp.float32) for _ in range(NBUF)],
            [pltpu.SemaphoreType.DMA for _ in range(NBUF)],
        ],
    )
    return f(idx, w, table)


# speedup vs baseline: 1.0437x; 1.0089x over previous
import functools

import jax
import jax.numpy as jnp
from jax import lax
from jax.experimental import pallas as pl
from jax.experimental.pallas import tpu as pltpu
from jax.experimental.pallas import tpu_sc as plsc

B = 16384
L = 26
D = 64
NL = 16
NC = 2
NS = 16
NW = NC * NS
BPW = B // NW
LP = 32
G = 4
GL = G * L
NG = BPW // G
DSL = D // NL


NBUF = 2


def _body(idx_hbm, w_hbm, table_hbm, out_hbm,
          idx_v, w_v, out_v, bufs, sems):
    c = lax.axis_index("c")
    s = lax.axis_index("s")
    wid = s * NC + c
    base = wid * BPW

    pltpu.sync_copy(idx_hbm.at[pl.ds(base * L, BPW * L)], idx_v)
    pltpu.sync_copy(w_hbm.at[pl.ds(base * LP, BPW * LP)], w_v)

    def start(step, buf, sem):
        pltpu.make_async_copy(
            table_hbm.at[idx_v.at[pl.ds(step * GL, GL)]], buf, sem).start()

    def wait(buf, sem):
        pltpu.make_async_copy(table_hbm.at[idx_v.at[pl.ds(0, GL)]],
                              buf, sem).wait()

    def compute(step, buf):
        for g in range(G):
            prow = step * G + g
            wv0 = w_v[pl.ds(prow * LP, NL)]
            wv1 = w_v[pl.ds(prow * LP + NL, NL)]
            accs = [jnp.zeros((NL,), jnp.float32) for _ in range(DSL)]
            for l in range(L):
                wl = wv0[l] if l < NL else wv1[l - NL]
                w = lax.broadcast(wl, (NL,))
                r = g * L + l
                for k in range(DSL):
                    accs[k] = accs[k] + w * buf[r, pl.ds(k * NL, NL)]
            for k in range(DSL):
                out_v[prow, pl.ds(k * NL, NL)] = accs[k]

    for k in range(NBUF - 1):
        start(k, bufs[k], sems[k])

    def loop_body(i, carry):
        for k in range(NBUF):
            step = NBUF * i + k
            nxt = step + NBUF - 1

            @pl.when(nxt < NG)
            def _():
                start(nxt, bufs[(k + NBUF - 1) % NBUF], sems[(k + NBUF - 1) % NBUF])

            wait(bufs[k], sems[k])
            compute(step, bufs[k])
        return carry

    lax.fori_loop(0, NG // NBUF, loop_body, 0)

    pltpu.sync_copy(out_v, out_hbm.at[pl.ds(base, BPW)])


V = 1000000
TBLK = 32768
NT = (V + TBLK - 1) // TBLK


VP = NT * TBLK


def _transpose_body(tT_ref, out_ref):
    m = tT_ref[...]
    half = TBLK // 2
    out_ref[:, :D] = jnp.transpose(m[:, :half], (1, 0))
    out_ref[:, D:] = jnp.transpose(m[:, half:], (1, 0))


def _linearize_table(table):
    t128 = pl.pallas_call(
        _transpose_body,
        grid=(NT,),
        in_specs=[pl.BlockSpec((D, TBLK), lambda i: (0, i))],
        out_specs=pl.BlockSpec((TBLK // 2, 2 * D), lambda i: (i, 0)),
        out_shape=jax.ShapeDtypeStruct((VP // 2, 2 * D), jnp.float32),
    )(table.T)
    return t128.reshape(VP, D)


def _remap_indices(idx):
    pos = idx % TBLK
    half = TBLK // 2
    return idx - pos + jnp.where(pos < half, 2 * pos, 2 * (pos - half) + 1)


@jax.jit
def kernel(feature_indices, feature_weights, table):
    idx = _remap_indices(feature_indices.reshape(-1).astype(jnp.int32))
    w = jnp.pad(feature_weights, ((0, 0), (0, LP - L))).reshape(-1)
    table = _linearize_table(table)

    mesh = plsc.VectorSubcoreMesh(core_axis_name="c", subcore_axis_name="s")
    f = pl.kernel(
        _body,
        out_type=jax.ShapeDtypeStruct((B, D), jnp.float32),
        mesh=mesh,
        compiler_params=pltpu.CompilerParams(use_tc_tiling_on_sc=False),
        scratch_types=[
            pltpu.VMEM((BPW * L,), jnp.int32),
            pltpu.VMEM((BPW * LP,), jnp.float32),
            pltpu.VMEM((BPW, D), jnp.float32),
            [pltpu.VMEM((GL, D), jnp.float32) for _ in range(NBUF)],
            [pltpu.SemaphoreType.DMA for _ in range(NBUF)],
        ],
    )
    return f(idx, w, table)
